# baseline (speedup 1.0000x reference)
import functools

import jax
import jax.numpy as jnp
from jax import lax
from jax.experimental import pallas as pl
from jax.experimental.pallas import tpu as pltpu
from jax.experimental.pallas import tpu_sc as plsc

_NC = 2
_NS = 16
_NW = _NC * _NS
_DW = 8
_VW = 8
_NB = 3


def _edge_chunk(ep, mult, align=1):
  for ch in range(128, 0, -1):
    if ep % ch == 0 and (ep // ch) % mult == 0 and ch % align == 0:
      return ch
  raise ValueError(f"no chunking for per-tile edge count {ep}")


def _sc_degree(dst, n, ch, kb):
  e = dst.shape[0]
  ep = e // _NW
  nchunk = ep // ch
  rpa = (n // _NS) // 8 * 8
  tail = n - _NS * rpa
  mesh = plsc.VectorSubcoreMesh(core_axis_name="c", subcore_axis_name="s")

  @functools.partial(
      pl.kernel,
      out_type=jax.ShapeDtypeStruct((_NC, n, _DW), jnp.float32),
      mesh=mesh,
      scratch_types=[
          pltpu.VMEM((ep,), jnp.int32),
          pltpu.VMEM((ch, _DW), jnp.float32),
          pltpu.VMEM_SHARED((n, _DW), jnp.float32),
          pltpu.SemaphoreType.DMA,
          pltpu.SemaphoreType.DMA,
      ],
      compiler_params=pltpu.CompilerParams(use_tc_tiling_on_sc=False),
  )
  def k(dst_hbm, ones_hbm, zero_hbm, out_hbm, didx, ones_v, acc, isem, ssem):
    c = lax.axis_index("c")
    s = lax.axis_index("s")
    wid = s * _NC + c
    ebase = pl.multiple_of(wid * ep, 8)
    rbase = pl.multiple_of(s * rpa, 8)
    ic = pltpu.async_copy(dst_hbm.at[pl.ds(ebase, ep)], didx, isem)
    pltpu.sync_copy(zero_hbm, acc.at[pl.ds(rbase, rpa)])
    if tail:
      @pl.when(s == _NS - 1)
      def _():
        pltpu.sync_copy(zero_hbm.at[pl.ds(0, tail)],
                        acc.at[pl.ds(_NS * rpa, tail)])
    pltpu.sync_copy(ones_hbm, ones_v)
    ic.wait()
    plsc.subcore_barrier()

    @pl.loop(0, nchunk // kb)
    def _(i):
      for k_ in range(kb):
        j = i * kb + k_
        pltpu.async_copy(ones_v, acc.at[didx.at[pl.ds(j * ch, ch)]], ssem,
                         add=True)
      for k_ in range(kb):
        j = i * kb + k_
        pltpu.make_async_copy(ones_v, acc.at[didx.at[pl.ds(j * ch, ch)]],
                              ssem).wait()

    plsc.subcore_barrier()
    pltpu.sync_copy(acc.at[pl.ds(rbase, rpa)], out_hbm.at[c, pl.ds(rbase, rpa)])
    if tail:
      @pl.when(s == _NS - 1)
      def _():
        pltpu.sync_copy(acc.at[pl.ds(_NS * rpa, tail)],
                        out_hbm.at[c, pl.ds(_NS * rpa, tail)])

  ones = jnp.ones((ch, _DW), jnp.float32)
  zero = jnp.zeros((rpa, _DW), jnp.float32)
  return k(dst, ones, zero)


def _sc_scatter(g, src, dst, ch):
  n, d = g.shape
  e = src.shape[0]
  ep = e // _NW
  nchunk = ep // ch
  rpa = (n // _NS) // 8 * 8
  tail = n - _NS * rpa
  mesh = plsc.VectorSubcoreMesh(core_axis_name="c", subcore_axis_name="s")

  @functools.partial(
      pl.kernel,
      out_type=jax.ShapeDtypeStruct((_NC, n, d), jnp.float32),
      mesh=mesh,
      scratch_types=[
          pltpu.VMEM((ep,), jnp.int32),
          pltpu.VMEM((ep,), jnp.int32),
          pltpu.VMEM((_NB, ch, d), jnp.float32),
          pltpu.VMEM_SHARED((n, d), jnp.float32),
          pltpu.SemaphoreType.DMA,
          pltpu.SemaphoreType.DMA,
          pltpu.SemaphoreType.DMA((2 * _NB,)),
      ],
      compiler_params=pltpu.CompilerParams(use_tc_tiling_on_sc=False),
  )
  def k(g_hbm, src_hbm, dst_hbm, zero_hbm, out_hbm, sidx, didx, rows, acc,
        is0, is1, sems):
    gsem = sems.at[pl.ds(0, _NB)]
    ssem = sems.at[pl.ds(_NB, _NB)]
    c = lax.axis_index("c")
    s = lax.axis_index("s")
    wid = s * _NC + c
    ebase = pl.multiple_of(wid * ep, 8)
    rbase = pl.multiple_of(s * rpa, 8)
    ic0 = pltpu.async_copy(src_hbm.at[pl.ds(ebase, ep)], sidx, is0)
    ic1 = pltpu.async_copy(dst_hbm.at[pl.ds(ebase, ep)], didx, is1)
    pltpu.sync_copy(zero_hbm, acc.at[pl.ds(rbase, rpa)])
    if tail:
      @pl.when(s == _NS - 1)
      def _():
        pltpu.sync_copy(zero_hbm.at[pl.ds(0, tail)],
                        acc.at[pl.ds(_NS * rpa, tail)])
    ic0.wait()
    ic1.wait()
    plsc.subcore_barrier()

    def g_start(j, b):
      pltpu.async_copy(g_hbm.at[sidx.at[pl.ds(j * ch, ch)]], rows.at[b],
                       gsem.at[b])

    def g_wait(j, b):
      pltpu.make_async_copy(g_hbm.at[sidx.at[pl.ds(j * ch, ch)]], rows.at[b],
                            gsem.at[b]).wait()

    def s_start(j, b):
      pltpu.async_copy(rows.at[b], acc.at[didx.at[pl.ds(j * ch, ch)]],
                       ssem.at[b], add=True)

    def s_wait(j, b):
      pltpu.make_async_copy(rows.at[b], acc.at[didx.at[pl.ds(j * ch, ch)]],
                            ssem.at[b]).wait()

    @pl.loop(0, nchunk + _NB)
    def _(t):
      b = lax.rem(t, _NB)
      bb = lax.rem(t + _NB - 1, _NB)

      @pl.when(t >= _NB)
      def _():
        s_wait(t - _NB, b)

      @pl.when(t < nchunk)
      def _():
        g_start(t, b)

      @pl.when(jnp.logical_and(t >= 1, t <= nchunk))
      def _():
        g_wait(t - 1, bb)
        s_start(t - 1, bb)

    plsc.subcore_barrier()
    pltpu.sync_copy(acc.at[pl.ds(rbase, rpa)],
                    out_hbm.at[c, pl.ds(rbase, rpa)])
    if tail:
      @pl.when(s == _NS - 1)
      def _():
        pltpu.sync_copy(acc.at[pl.ds(_NS * rpa, tail)],
                        out_hbm.at[c, pl.ds(_NS * rpa, tail)])

  zero = jnp.zeros((rpa, d), jnp.float32)
  return k(g, src, dst, zero)


def _tc_prep(degp, x, w1, bn):
  n, d = x.shape

  def body(degp_ref, x_ref, w1_ref, g1_ref, dinv_ref):
    deg = degp_ref[0, :, 0:1] + degp_ref[1, :, 0:1] + 1.0
    dv = lax.rsqrt(deg)
    hx = jnp.dot(x_ref[...], w1_ref[...], preferred_element_type=jnp.float32)
    g1_ref[...] = hx * dv
    dinv_ref[...] = jnp.broadcast_to(dv, (bn, _VW))

  grid = n // bn
  return pl.pallas_call(
      body,
      grid=(grid,),
      in_specs=[
          pl.BlockSpec((_NC, bn, _DW), lambda i: (0, i, 0)),
          pl.BlockSpec((bn, d), lambda i: (i, 0)),
          pl.BlockSpec((d, d), lambda i: (0, 0)),
      ],
      out_specs=[
          pl.BlockSpec((bn, d), lambda i: (i, 0)),
          pl.BlockSpec((bn, _VW), lambda i: (i, 0)),
      ],
      out_shape=[
          jax.ShapeDtypeStruct((n, d), jnp.float32),
          jax.ShapeDtypeStruct((n, _VW), jnp.float32),
      ],
  )(degp, x, w1)


def _tc_mid(sp, g1, dinv, b1, w2, bn):
  n, d = g1.shape

  def body(sp_ref, g1_ref, dinv_ref, b1_ref, w2_ref, g2_ref):
    dv = dinv_ref[:, 0:1]
    h1 = dv * (sp_ref[0] + sp_ref[1] + g1_ref[...]) + b1_ref[...]
    h1 = jnp.maximum(h1, 0.0)
    g2_ref[...] = jnp.dot(h1, w2_ref[...], preferred_element_type=jnp.float32) * dv

  grid = n // bn
  return pl.pallas_call(
      body,
      grid=(grid,),
      in_specs=[
          pl.BlockSpec((_NC, bn, d), lambda i: (0, i, 0)),
          pl.BlockSpec((bn, d), lambda i: (i, 0)),
          pl.BlockSpec((bn, _VW), lambda i: (i, 0)),
          pl.BlockSpec((1, d), lambda i: (0, 0)),
          pl.BlockSpec((d, d), lambda i: (0, 0)),
      ],
      out_specs=pl.BlockSpec((bn, d), lambda i: (i, 0)),
      out_shape=jax.ShapeDtypeStruct((n, d), jnp.float32),
  )(sp, g1, dinv, b1.reshape(1, d), w2)


def _tc_out(sp, g2, dinv, b2, wp, bp, bn):
  n, d = g2.shape
  dp = wp.shape[1]

  def body(sp_ref, g2_ref, dinv_ref, b2_ref, wp_ref, bp_ref, out_ref):
    dv = dinv_ref[:, 0:1]
    h2 = dv * (sp_ref[0] + sp_ref[1] + g2_ref[...]) + b2_ref[...]
    out_ref[...] = (
        jnp.dot(h2, wp_ref[...], preferred_element_type=jnp.float32)
        + bp_ref[...]
    )

  grid = n // bn
  return pl.pallas_call(
      body,
      grid=(grid,),
      in_specs=[
          pl.BlockSpec((_NC, bn, d), lambda i: (0, i, 0)),
          pl.BlockSpec((bn, d), lambda i: (i, 0)),
          pl.BlockSpec((bn, _VW), lambda i: (i, 0)),
          pl.BlockSpec((1, d), lambda i: (0, 0)),
          pl.BlockSpec((d, dp), lambda i: (0, 0)),
          pl.BlockSpec((1, dp), lambda i: (0, 0)),
      ],
      out_specs=pl.BlockSpec((bn, dp), lambda i: (i, 0)),
      out_shape=jax.ShapeDtypeStruct((n, dp), jnp.float32),
  )(sp, g2, dinv, b2.reshape(1, d), wp, bp.reshape(1, dp))


def kernel(x, edge_index, W1, b1, W2, b2, Wp, bp):
  n = x.shape[0]
  e = edge_index.shape[1]
  ep = e // _NW
  ch = _edge_chunk(ep, 1, align=8)
  kb = 5 if (ep // ch) % 5 == 0 else 1
  bn = 2000 if n % 2000 == 0 else (1000 if n % 1000 == 0 else 8)

  src = edge_index[0]
  dst = edge_index[1]
  degp = _sc_degree(dst, n, ch, kb)
  g1, dinv = _tc_prep(degp, x, W1, bn)
  s1 = _sc_scatter(g1, src, dst, ch)
  g2 = _tc_mid(s1, g1, dinv, b1, W2, bn)
  s2 = _sc_scatter(g2, src, dst, ch)
  return _tc_out(s2, g2, dinv, b2, Wp, bp, bn)


# speedup vs baseline: 1.0422x; 1.0422x over previous
import functools

import jax
import jax.numpy as jnp
from jax import lax
from jax.experimental import pallas as pl
from jax.experimental.pallas import tpu as pltpu
from jax.experimental.pallas import tpu_sc as plsc

_NC = 2
_NS = 16
_NW = _NC * _NS
_DW = 8
_VW = 8
_NB = 3


def _edge_chunk(ep, mult, align=1):
  for ch in range(128, 0, -1):
    if ep % ch == 0 and (ep // ch) % mult == 0 and ch % align == 0:
      return ch
  raise ValueError(f"no chunking for per-tile edge count {ep}")


def _sc_degree(ei, n, ch, kb):
  e = ei.shape[1]
  ep = e // _NW
  nchunk = ep // ch
  rpa = (n // _NS) // 8 * 8
  tail = n - _NS * rpa
  mesh = plsc.VectorSubcoreMesh(core_axis_name="c", subcore_axis_name="s")

  @functools.partial(
      pl.kernel,
      out_type=jax.ShapeDtypeStruct((_NC, n, _DW), jnp.float32),
      mesh=mesh,
      scratch_types=[
          pltpu.VMEM((ep,), jnp.int32),
          pltpu.VMEM((ch, _DW), jnp.float32),
          pltpu.VMEM_SHARED((n, _DW), jnp.float32),
          pltpu.SemaphoreType.DMA,
          pltpu.SemaphoreType.DMA,
      ],
      compiler_params=pltpu.CompilerParams(use_tc_tiling_on_sc=False),
  )
  def k(ei_hbm, ones_hbm, zero_hbm, out_hbm, didx, ones_v, acc, isem, ssem):
    c = lax.axis_index("c")
    s = lax.axis_index("s")
    wid = s * _NC + c
    ebase = pl.multiple_of(wid * ep, 8)
    rbase = pl.multiple_of(s * rpa, 8)
    ic = pltpu.async_copy(ei_hbm.at[1, pl.ds(ebase, ep)], didx, isem)
    pltpu.sync_copy(zero_hbm, acc.at[pl.ds(rbase, rpa)])
    if tail:
      @pl.when(s == _NS - 1)
      def _():
        pltpu.sync_copy(zero_hbm.at[pl.ds(0, tail)],
                        acc.at[pl.ds(_NS * rpa, tail)])
    pltpu.sync_copy(ones_hbm, ones_v)
    ic.wait()
    plsc.subcore_barrier()

    @pl.loop(0, nchunk // kb)
    def _(i):
      for k_ in range(kb):
        j = i * kb + k_
        pltpu.async_copy(ones_v, acc.at[didx.at[pl.ds(j * ch, ch)]], ssem,
                         add=True)
      for k_ in range(kb):
        j = i * kb + k_
        pltpu.make_async_copy(ones_v, acc.at[didx.at[pl.ds(j * ch, ch)]],
                              ssem).wait()

    plsc.subcore_barrier()
    pltpu.sync_copy(acc.at[pl.ds(rbase, rpa)], out_hbm.at[c, pl.ds(rbase, rpa)])
    if tail:
      @pl.when(s == _NS - 1)
      def _():
        pltpu.sync_copy(acc.at[pl.ds(_NS * rpa, tail)],
                        out_hbm.at[c, pl.ds(_NS * rpa, tail)])

  ones = jnp.ones((ch, _DW), jnp.float32)
  zero = jnp.zeros((rpa, _DW), jnp.float32)
  return k(ei, ones, zero)


def _sc_scatter(g, ei, ch):
  n, d = g.shape
  e = ei.shape[1]
  ep = e // _NW
  nchunk = ep // ch
  rpa = (n // _NS) // 8 * 8
  tail = n - _NS * rpa
  mesh = plsc.VectorSubcoreMesh(core_axis_name="c", subcore_axis_name="s")

  @functools.partial(
      pl.kernel,
      out_type=jax.ShapeDtypeStruct((_NC, n, d), jnp.float32),
      mesh=mesh,
      scratch_types=[
          pltpu.VMEM((ep,), jnp.int32),
          pltpu.VMEM((ep,), jnp.int32),
          pltpu.VMEM((_NB, ch, d), jnp.float32),
          pltpu.VMEM_SHARED((n, d), jnp.float32),
          pltpu.SemaphoreType.DMA,
          pltpu.SemaphoreType.DMA,
          pltpu.SemaphoreType.DMA((2 * _NB,)),
      ],
      compiler_params=pltpu.CompilerParams(use_tc_tiling_on_sc=False),
  )
  def k(g_hbm, ei_hbm, zero_hbm, out_hbm, sidx, didx, rows, acc,
        is0, is1, sems):
    gsem = sems.at[pl.ds(0, _NB)]
    ssem = sems.at[pl.ds(_NB, _NB)]
    c = lax.axis_index("c")
    s = lax.axis_index("s")
    wid = s * _NC + c
    ebase = pl.multiple_of(wid * ep, 8)
    rbase = pl.multiple_of(s * rpa, 8)
    ic0 = pltpu.async_copy(ei_hbm.at[0, pl.ds(ebase, ep)], sidx, is0)
    ic1 = pltpu.async_copy(ei_hbm.at[1, pl.ds(ebase, ep)], didx, is1)
    pltpu.sync_copy(zero_hbm, acc.at[pl.ds(rbase, rpa)])
    if tail:
      @pl.when(s == _NS - 1)
      def _():
        pltpu.sync_copy(zero_hbm.at[pl.ds(0, tail)],
                        acc.at[pl.ds(_NS * rpa, tail)])
    ic0.wait()
    ic1.wait()
    plsc.subcore_barrier()

    def g_start(j, b):
      pltpu.async_copy(g_hbm.at[sidx.at[pl.ds(j * ch, ch)]], rows.at[b],
                       gsem.at[b])

    def g_wait(j, b):
      pltpu.make_async_copy(g_hbm.at[sidx.at[pl.ds(j * ch, ch)]], rows.at[b],
                            gsem.at[b]).wait()

    def s_start(j, b):
      pltpu.async_copy(rows.at[b], acc.at[didx.at[pl.ds(j * ch, ch)]],
                       ssem.at[b], add=True)

    def s_wait(j, b):
      pltpu.make_async_copy(rows.at[b], acc.at[didx.at[pl.ds(j * ch, ch)]],
                            ssem.at[b]).wait()

    @pl.loop(0, nchunk + _NB)
    def _(t):
      b = lax.rem(t, _NB)
      bb = lax.rem(t + _NB - 1, _NB)

      @pl.when(t >= _NB)
      def _():
        s_wait(t - _NB, b)

      @pl.when(t < nchunk)
      def _():
        g_start(t, b)

      @pl.when(jnp.logical_and(t >= 1, t <= nchunk))
      def _():
        g_wait(t - 1, bb)
        s_start(t - 1, bb)

    plsc.subcore_barrier()
    pltpu.sync_copy(acc.at[pl.ds(rbase, rpa)],
                    out_hbm.at[c, pl.ds(rbase, rpa)])
    if tail:
      @pl.when(s == _NS - 1)
      def _():
        pltpu.sync_copy(acc.at[pl.ds(_NS * rpa, tail)],
                        out_hbm.at[c, pl.ds(_NS * rpa, tail)])

  zero = jnp.zeros((rpa, d), jnp.float32)
  return k(g, ei, zero)


def _tc_prep(degp, x, w1, bn):
  n, d = x.shape

  def body(degp_ref, x_ref, w1_ref, g1_ref, dinv_ref):
    deg = degp_ref[0, :, 0:1] + degp_ref[1, :, 0:1] + 1.0
    dv = lax.rsqrt(deg)
    hx = jnp.dot(x_ref[...], w1_ref[...], preferred_element_type=jnp.float32)
    g1_ref[...] = hx * dv
    dinv_ref[...] = jnp.broadcast_to(dv, (bn, _VW))

  grid = n // bn
  return pl.pallas_call(
      body,
      grid=(grid,),
      in_specs=[
          pl.BlockSpec((_NC, bn, _DW), lambda i: (0, i, 0)),
          pl.BlockSpec((bn, d), lambda i: (i, 0)),
          pl.BlockSpec((d, d), lambda i: (0, 0)),
      ],
      out_specs=[
          pl.BlockSpec((bn, d), lambda i: (i, 0)),
          pl.BlockSpec((bn, _VW), lambda i: (i, 0)),
      ],
      out_shape=[
          jax.ShapeDtypeStruct((n, d), jnp.float32),
          jax.ShapeDtypeStruct((n, _VW), jnp.float32),
      ],
  )(degp, x, w1)


def _tc_mid(sp, g1, dinv, b1, w2, bn):
  n, d = g1.shape

  def body(sp_ref, g1_ref, dinv_ref, b1_ref, w2_ref, g2_ref):
    dv = dinv_ref[:, 0:1]
    h1 = dv * (sp_ref[0] + sp_ref[1] + g1_ref[...]) + b1_ref[...]
    h1 = jnp.maximum(h1, 0.0)
    g2_ref[...] = jnp.dot(h1, w2_ref[...], preferred_element_type=jnp.float32) * dv

  grid = n // bn
  return pl.pallas_call(
      body,
      grid=(grid,),
      in_specs=[
          pl.BlockSpec((_NC, bn, d), lambda i: (0, i, 0)),
          pl.BlockSpec((bn, d), lambda i: (i, 0)),
          pl.BlockSpec((bn, _VW), lambda i: (i, 0)),
          pl.BlockSpec((1, d), lambda i: (0, 0)),
          pl.BlockSpec((d, d), lambda i: (0, 0)),
      ],
      out_specs=pl.BlockSpec((bn, d), lambda i: (i, 0)),
      out_shape=jax.ShapeDtypeStruct((n, d), jnp.float32),
  )(sp, g1, dinv, b1.reshape(1, d), w2)


def _tc_out(sp, g2, dinv, b2, wp, bp, bn):
  n, d = g2.shape
  dp = wp.shape[1]

  def body(sp_ref, g2_ref, dinv_ref, b2_ref, wp_ref, bp_ref, out_ref):
    dv = dinv_ref[:, 0:1]
    h2 = dv * (sp_ref[0] + sp_ref[1] + g2_ref[...]) + b2_ref[...]
    out_ref[...] = (
        jnp.dot(h2, wp_ref[...], preferred_element_type=jnp.float32)
        + bp_ref[...]
    )

  grid = n // bn
  return pl.pallas_call(
      body,
      grid=(grid,),
      in_specs=[
          pl.BlockSpec((_NC, bn, d), lambda i: (0, i, 0)),
          pl.BlockSpec((bn, d), lambda i: (i, 0)),
          pl.BlockSpec((bn, _VW), lambda i: (i, 0)),
          pl.BlockSpec((1, d), lambda i: (0, 0)),
          pl.BlockSpec((d, dp), lambda i: (0, 0)),
          pl.BlockSpec((1, dp), lambda i: (0, 0)),
      ],
      out_specs=pl.BlockSpec((bn, dp), lambda i: (i, 0)),
      out_shape=jax.ShapeDtypeStruct((n, dp), jnp.float32),
  )(sp, g2, dinv, b2.reshape(1, d), wp, bp.reshape(1, dp))


def kernel(x, edge_index, W1, b1, W2, b2, Wp, bp):
  n = x.shape[0]
  e = edge_index.shape[1]
  ep = e // _NW
  ch = _edge_chunk(ep, 1, align=8)
  kb = 5 if (ep // ch) % 5 == 0 else 1
  bn = 2000 if n % 2000 == 0 else (1000 if n % 1000 == 0 else 8)

  degp = _sc_degree(edge_index, n, ch, kb)
  g1, dinv = _tc_prep(degp, x, W1, bn)
  s1 = _sc_scatter(g1, edge_index, ch)
  g2 = _tc_mid(s1, g1, dinv, b1, W2, bn)
  s2 = _sc_scatter(g2, edge_index, ch)
  return _tc_out(s2, g2, dinv, b2, Wp, bp, bn)
